# Initial kernel scaffold; baseline (speedup 1.0000x reference)
import jax
import jax.numpy as jnp
from jax.experimental import pallas as pl


def kernel(node_features, edge_index, params):
    raise NotImplementedError("write your pallas kernel here")


# speedup vs baseline: 18.8250x; 18.8250x over previous
import functools

import jax
import jax.numpy as jnp
from jax import lax
from jax.experimental import pallas as pl
from jax.experimental.pallas import tpu as pltpu
from jax.experimental.pallas import tpu_sc as plsc

_N = 10000
_E = 320000
_H = 128
_NUM_STATIONS = 10000

_NW = 32
_C = 80
_CH_TOT = _E // _C
_NCH_W = _CH_TOT // _NW
_RPT = 624
_TAIL = _N - 16 * _RPT
_TAILBASE = 16 * _RPT

_R = 1000
_CB = 2048

_sc_mesh = plsc.VectorSubcoreMesh(core_axis_name="c", subcore_axis_name="s")


@functools.partial(
    pl.kernel,
    mesh=_sc_mesh,
    out_type=jax.ShapeDtypeStruct((2, _N, 16), jnp.float32),
    scratch_types=[
        pltpu.VMEM((_NCH_W, _C), jnp.int32),
        pltpu.VMEM((_C, 16), jnp.float32),
        pltpu.VMEM_SHARED((_N, 16), jnp.float32),
    ],
)
def _deg_kernel(ones_hbm, dst_hbm, out_hbm, didx, onesv, acc):
    cid = lax.axis_index("c")
    sid = lax.axis_index("s")
    wid = cid * 16 + sid
    base = sid * _RPT
    pltpu.sync_copy(ones_hbm.at[pl.ds(base, _RPT)], acc.at[pl.ds(base, _RPT)])

    @pl.when(sid == 15)
    def _():
        pltpu.sync_copy(ones_hbm.at[pl.ds(_TAILBASE, _TAIL)],
                        acc.at[pl.ds(_TAILBASE, _TAIL)])

    pltpu.sync_copy(dst_hbm.at[wid], didx)
    pltpu.sync_copy(ones_hbm.at[pl.ds(0, _C)], onesv)
    plsc.subcore_barrier()

    def body(j, carry):
        pltpu.sync_copy(onesv, acc.at[didx.at[j]], add=True)
        return carry

    lax.fori_loop(0, _NCH_W, body, 0)
    plsc.subcore_barrier()
    pltpu.sync_copy(acc.at[pl.ds(base, _RPT)],
                    out_hbm.at[cid, pl.ds(base, _RPT)])

    @pl.when(sid == 15)
    def _():
        pltpu.sync_copy(acc.at[pl.ds(_TAILBASE, _TAIL)],
                        out_hbm.at[cid, pl.ds(_TAILBASE, _TAIL)])


@functools.partial(
    pl.kernel,
    mesh=_sc_mesh,
    out_type=jax.ShapeDtypeStruct((2, _N, _H), jnp.float32),
    scratch_types=[
        pltpu.VMEM((_NCH_W, _C), jnp.int32),
        pltpu.VMEM((_NCH_W, _C), jnp.int32),
        pltpu.VMEM((_C, _H), jnp.float32),
        pltpu.VMEM_SHARED((_N, _H), jnp.float32),
        pltpu.SemaphoreType.DMA,
    ],
)
def _edge_kernel(z_hbm, src_hbm, dst_hbm, out_hbm, sidx, didx, rows, acc, sem):
    cid = lax.axis_index("c")
    sid = lax.axis_index("s")
    wid = cid * 16 + sid
    base = sid * _RPT
    pltpu.sync_copy(z_hbm.at[pl.ds(base, _RPT)], acc.at[pl.ds(base, _RPT)])

    @pl.when(sid == 15)
    def _():
        pltpu.sync_copy(z_hbm.at[pl.ds(_TAILBASE, _TAIL)],
                        acc.at[pl.ds(_TAILBASE, _TAIL)])

    pltpu.sync_copy(src_hbm.at[wid], sidx)
    pltpu.sync_copy(dst_hbm.at[wid], didx)
    plsc.subcore_barrier()

    def body(j, carry):
        pltpu.async_copy(z_hbm.at[sidx.at[j]], rows, sem).wait()
        pltpu.sync_copy(rows, acc.at[didx.at[j]], add=True)
        return carry

    lax.fori_loop(0, _NCH_W, body, 0)
    plsc.subcore_barrier()
    pltpu.sync_copy(acc.at[pl.ds(base, _RPT)],
                    out_hbm.at[cid, pl.ds(base, _RPT)])

    @pl.when(sid == 15)
    def _():
        pltpu.sync_copy(acc.at[pl.ds(_TAILBASE, _TAIL)],
                        out_hbm.at[cid, pl.ds(_TAILBASE, _TAIL)])


def _deg_dinv(degp_ref):
    deg = (jnp.max(degp_ref[0], axis=-1) + jnp.max(degp_ref[1], axis=-1)
           - 1.0)
    return lax.rsqrt(deg)


def _tc1_body(x_ref, wp_ref, bp_ref, w1_ref, degp_ref, z_ref):
    h = jnp.maximum(
        jnp.dot(x_ref[...], wp_ref[...], preferred_element_type=jnp.float32)
        + bp_ref[...], 0.0)
    dinv = _deg_dinv(degp_ref)
    z_ref[...] = jnp.dot(h, w1_ref[...],
                         preferred_element_type=jnp.float32) * dinv[:, None]


def _tc1(x, wp, bp, w1, degp):
    return pl.pallas_call(
        _tc1_body,
        grid=(_N // _R,),
        in_specs=[
            pl.BlockSpec((_R, _H), lambda i: (i, 0)),
            pl.BlockSpec((_H, _H), lambda i: (0, 0)),
            pl.BlockSpec((1, _H), lambda i: (0, 0)),
            pl.BlockSpec((_H, _H), lambda i: (0, 0)),
            pl.BlockSpec((2, _R, 16), lambda i: (0, i, 0)),
        ],
        out_specs=pl.BlockSpec((_R, _H), lambda i: (i, 0)),
        out_shape=jax.ShapeDtypeStruct((_N, _H), jnp.float32),
    )(x, wp, bp, w1, degp)


def _tc2_body(aggp_ref, z1_ref, degp_ref, b1_ref, w2_ref, z2_ref):
    dinv = _deg_dinv(degp_ref)
    s = aggp_ref[0] + aggp_ref[1] - z1_ref[...]
    h1 = jnp.maximum(s * dinv[:, None] + b1_ref[...], 0.0)
    z2_ref[...] = jnp.dot(h1, w2_ref[...],
                          preferred_element_type=jnp.float32) * dinv[:, None]


def _tc2(aggp, z1, degp, b1, w2):
    return pl.pallas_call(
        _tc2_body,
        grid=(_N // _R,),
        in_specs=[
            pl.BlockSpec((2, _R, _H), lambda i: (0, i, 0)),
            pl.BlockSpec((_R, _H), lambda i: (i, 0)),
            pl.BlockSpec((2, _R, 16), lambda i: (0, i, 0)),
            pl.BlockSpec((1, _H), lambda i: (0, 0)),
            pl.BlockSpec((_H, _H), lambda i: (0, 0)),
        ],
        out_specs=pl.BlockSpec((_R, _H), lambda i: (i, 0)),
        out_shape=jax.ShapeDtypeStruct((_N, _H), jnp.float32),
    )(aggp, z1, degp, b1, w2)


def _tc3_body(aggp_ref, z2_ref, degp_ref, b2_ref, g_ref):
    i = pl.program_id(0)
    dinv = _deg_dinv(degp_ref)
    s = aggp_ref[0] + aggp_ref[1] - z2_ref[...]
    h2 = jnp.maximum(s * dinv[:, None] + b2_ref[...], 0.0)
    bsum = jnp.sum(h2, axis=0, keepdims=True)

    @pl.when(i == 0)
    def _():
        g_ref[...] = jnp.zeros_like(g_ref)

    g_ref[...] += bsum


def _tc3(aggp, z2, degp, b2):
    return pl.pallas_call(
        _tc3_body,
        grid=(_N // _R,),
        in_specs=[
            pl.BlockSpec((2, _R, _H), lambda i: (0, i, 0)),
            pl.BlockSpec((_R, _H), lambda i: (i, 0)),
            pl.BlockSpec((2, _R, 16), lambda i: (0, i, 0)),
            pl.BlockSpec((1, _H), lambda i: (0, 0)),
        ],
        out_specs=pl.BlockSpec((1, _H), lambda i: (0, 0)),
        out_shape=jax.ShapeDtypeStruct((1, _H), jnp.float32),
    )(aggp, z2, degp, b2)


def _tc4_body(g_ref, wc_ref, bc_ref, whl_ref, bhl_ref, wmlt_ref, bmlt_ref,
              wsl_ref, bsl_ref, wp1_ref, bp1_ref, wp2_ref, bp2_ref,
              wdt_ref, bdt_ref,
              value_ref, hl_ref, mlt_ref, p1_ref, p2_ref, dt_ref, sl_ref):
    j = pl.program_id(0)
    g = g_ref[...] * (1.0 / _N)

    @pl.when(j == 0)
    def _():
        value_ref[...] = jnp.dot(
            g, wc_ref[...], preferred_element_type=jnp.float32) + bc_ref[...]
        hl_ref[...] = jnp.dot(
            g, whl_ref[...], preferred_element_type=jnp.float32) + bhl_ref[...]
        mlt_ref[...] = jnp.dot(
            g, wmlt_ref[...],
            preferred_element_type=jnp.float32) + bmlt_ref[...]
        sl_ref[...] = jnp.dot(
            g, wsl_ref[...], preferred_element_type=jnp.float32) + bsl_ref[...]

    p1_ref[...] = jnp.dot(
        g, wp1_ref[...], preferred_element_type=jnp.float32) + bp1_ref[...]
    p2_ref[...] = jnp.dot(
        g, wp2_ref[...], preferred_element_type=jnp.float32) + bp2_ref[...]
    dt_ref[...] = jnp.dot(
        g, wdt_ref[...], preferred_element_type=jnp.float32) + bdt_ref[...]


def _tc4(gsum, p):
    small = lambda w: pl.BlockSpec(w, lambda j: (0, 0))
    bigw = pl.BlockSpec((_H, _CB), lambda j: (0, j))
    bigb = pl.BlockSpec((1, _CB), lambda j: (0, j))
    return pl.pallas_call(
        _tc4_body,
        grid=(pl.cdiv(_NUM_STATIONS, _CB),),
        in_specs=[
            small((1, _H)),
            small((_H, 1)), small((1, 1)),
            small((_H, 4)), small((1, 4)),
            small((_H, 3)), small((1, 3)),
            small((_H, 8)), small((1, 8)),
            bigw, bigb,
            bigw, bigb,
            bigw, bigb,
        ],
        out_specs=[
            small((1, 1)), small((1, 4)), small((1, 3)),
            pl.BlockSpec((1, _CB), lambda j: (0, j)),
            pl.BlockSpec((1, _CB), lambda j: (0, j)),
            pl.BlockSpec((1, _CB), lambda j: (0, j)),
            small((1, 8)),
        ],
        out_shape=[
            jax.ShapeDtypeStruct((1, 1), jnp.float32),
            jax.ShapeDtypeStruct((1, 4), jnp.float32),
            jax.ShapeDtypeStruct((1, 3), jnp.float32),
            jax.ShapeDtypeStruct((1, _NUM_STATIONS), jnp.float32),
            jax.ShapeDtypeStruct((1, _NUM_STATIONS), jnp.float32),
            jax.ShapeDtypeStruct((1, _NUM_STATIONS), jnp.float32),
            jax.ShapeDtypeStruct((1, 8), jnp.float32),
        ],
    )(gsum,
      p['W_critic'], p['b_critic'][None, :],
      p['W_hl'], p['b_hl'][None, :],
      p['W_mlt'], p['b_mlt'][None, :],
      p['W_sl'], p['b_sl'][None, :],
      p['W_p1'], p['b_p1'][None, :],
      p['W_p2'], p['b_p2'][None, :],
      p['W_dt'], p['b_dt'][None, :])


def kernel(node_features, edge_index, params):
    p = params
    src2 = edge_index[0].astype(jnp.int32).reshape(_NW, _NCH_W, _C)
    dst2 = edge_index[1].astype(jnp.int32).reshape(_NW, _NCH_W, _C)
    ones16 = jnp.ones((_N, 16), jnp.float32)

    degp = _deg_kernel(ones16, dst2)
    z1 = _tc1(node_features, p['W_proj'], p['b_proj'][None, :], p['W1'], degp)
    agg1 = _edge_kernel(z1, src2, dst2)
    z2 = _tc2(agg1, z1, degp, p['b1'][None, :], p['W2'])
    agg2 = _edge_kernel(z2, src2, dst2)
    gsum = _tc3(agg2, z2, degp, p['b2'][None, :])
    value, hl, mlt, p1, p2, dt, sl = _tc4(gsum, p)
    return (value, hl, mlt, p1, p2, dt, sl)


# speedup vs baseline: 23.5303x; 1.2500x over previous
import functools

import jax
import jax.numpy as jnp
from jax import lax
from jax.experimental import pallas as pl
from jax.experimental.pallas import tpu as pltpu
from jax.experimental.pallas import tpu_sc as plsc

_N = 10000
_E = 320000
_H = 128
_NUM_STATIONS = 10000

_NW = 32
_C = 80
_NCH_W = _E // _NW // _C
_RPT = 624
_TAIL = _N - 16 * _RPT
_TAILBASE = 16 * _RPT

_R = 1000
_CB = 2048

_sc_mesh = plsc.VectorSubcoreMesh(core_axis_name="c", subcore_axis_name="s")


@functools.partial(
    pl.kernel,
    mesh=_sc_mesh,
    out_type=jax.ShapeDtypeStruct((2, _N, 16), jnp.float32),
    scratch_types=[
        pltpu.VMEM((_NCH_W, _C), jnp.int32),
        pltpu.VMEM((_C, 16), jnp.float32),
        pltpu.VMEM_SHARED((_N, 16), jnp.float32),
        pltpu.SemaphoreType.DMA,
    ],
)
def _deg_kernel(ones_hbm, dst_hbm, out_hbm, didx, onesv, acc, ssem):
    cid = lax.axis_index("c")
    sid = lax.axis_index("s")
    wid = cid * 16 + sid
    base = sid * _RPT
    pltpu.sync_copy(ones_hbm.at[pl.ds(base, _RPT)], acc.at[pl.ds(base, _RPT)])

    @pl.when(sid == 15)
    def _():
        pltpu.sync_copy(ones_hbm.at[pl.ds(_TAILBASE, _TAIL)],
                        acc.at[pl.ds(_TAILBASE, _TAIL)])

    pltpu.sync_copy(dst_hbm.at[wid], didx)
    pltpu.sync_copy(ones_hbm.at[pl.ds(0, _C)], onesv)
    plsc.subcore_barrier()

    def body(j, carry):
        pltpu.async_copy(onesv, acc.at[didx.at[j]], ssem, add=True)

        @pl.when(j >= 4)
        def _():
            pltpu.make_async_copy(onesv, acc.at[didx.at[j - 4]], ssem).wait()

        return carry

    lax.fori_loop(0, _NCH_W, body, 0)

    def drain(j, carry):
        pltpu.make_async_copy(onesv, acc.at[didx.at[j]], ssem).wait()
        return carry

    lax.fori_loop(_NCH_W - 4, _NCH_W, drain, 0)
    plsc.subcore_barrier()
    pltpu.sync_copy(acc.at[pl.ds(base, _RPT)],
                    out_hbm.at[cid, pl.ds(base, _RPT)])

    @pl.when(sid == 15)
    def _():
        pltpu.sync_copy(acc.at[pl.ds(_TAILBASE, _TAIL)],
                        out_hbm.at[cid, pl.ds(_TAILBASE, _TAIL)])


@functools.partial(
    pl.kernel,
    mesh=_sc_mesh,
    out_type=jax.ShapeDtypeStruct((2, _N, _H), jnp.float32),
    scratch_types=[
        pltpu.VMEM((3, 2, _C), jnp.int32),
        pltpu.VMEM((2, _C, _H), jnp.float32),
        pltpu.VMEM_SHARED((_N, _H), jnp.float32),
        pltpu.SemaphoreType.DMA,
        pltpu.SemaphoreType.DMA,
        pltpu.SemaphoreType.DMA,
    ],
)
def _edge_kernel(z_hbm, e_hbm, out_hbm, idx, rows, acc, isem, gsem, ssem):
    cid = lax.axis_index("c")
    sid = lax.axis_index("s")
    wid = cid * 16 + sid
    base = sid * _RPT
    pltpu.sync_copy(z_hbm.at[pl.ds(base, _RPT)], acc.at[pl.ds(base, _RPT)])

    @pl.when(sid == 15)
    def _():
        pltpu.sync_copy(z_hbm.at[pl.ds(_TAILBASE, _TAIL)],
                        acc.at[pl.ds(_TAILBASE, _TAIL)])

    plsc.subcore_barrier()

    pltpu.async_copy(e_hbm.at[wid, 0], idx.at[0], isem)
    pltpu.async_copy(e_hbm.at[wid, 1], idx.at[1], isem)
    pltpu.make_async_copy(e_hbm.at[wid, 0], idx.at[0], isem).wait()
    pltpu.async_copy(z_hbm.at[idx.at[0, 0]], rows.at[0], gsem)

    def body(j, carry):
        s3 = lax.rem(j, 3)
        s2 = lax.rem(j, 2)
        pltpu.make_async_copy(z_hbm.at[idx.at[s3, 0]], rows.at[s2],
                              gsem).wait()

        @pl.when(j >= 1)
        def _():
            pltpu.make_async_copy(rows.at[1 - s2],
                                  acc.at[idx.at[lax.rem(j + 2, 3), 1]],
                                  ssem).wait()

        @pl.when(j < _NCH_W - 1)
        def _():
            n3 = lax.rem(j + 1, 3)
            pltpu.make_async_copy(e_hbm.at[wid, j + 1], idx.at[n3],
                                  isem).wait()
            pltpu.async_copy(z_hbm.at[idx.at[n3, 0]], rows.at[1 - s2], gsem)

        @pl.when(j < _NCH_W - 2)
        def _():
            pltpu.async_copy(e_hbm.at[wid, j + 2], idx.at[lax.rem(j + 2, 3)],
                             isem)

        pltpu.async_copy(rows.at[s2], acc.at[idx.at[s3, 1]], ssem, add=True)
        return carry

    lax.fori_loop(0, _NCH_W, body, 0)
    pltpu.make_async_copy(rows.at[(_NCH_W - 1) % 2],
                          acc.at[idx.at[(_NCH_W - 1) % 3, 1]], ssem).wait()
    plsc.subcore_barrier()
    pltpu.sync_copy(acc.at[pl.ds(base, _RPT)],
                    out_hbm.at[cid, pl.ds(base, _RPT)])

    @pl.when(sid == 15)
    def _():
        pltpu.sync_copy(acc.at[pl.ds(_TAILBASE, _TAIL)],
                        out_hbm.at[cid, pl.ds(_TAILBASE, _TAIL)])


def _deg_dinv(degp_ref):
    deg = (jnp.max(degp_ref[0], axis=-1) + jnp.max(degp_ref[1], axis=-1)
           - 1.0)
    return lax.rsqrt(deg)


def _tc1_body(x_ref, wp_ref, bp_ref, w1_ref, degp_ref, z_ref):
    h = jnp.maximum(
        jnp.dot(x_ref[...], wp_ref[...], preferred_element_type=jnp.float32)
        + bp_ref[...], 0.0)
    dinv = _deg_dinv(degp_ref)
    z_ref[...] = jnp.dot(h, w1_ref[...],
                         preferred_element_type=jnp.float32) * dinv[:, None]


def _tc1(x, wp, bp, w1, degp):
    return pl.pallas_call(
        _tc1_body,
        grid=(_N // _R,),
        in_specs=[
            pl.BlockSpec((_R, _H), lambda i: (i, 0)),
            pl.BlockSpec((_H, _H), lambda i: (0, 0)),
            pl.BlockSpec((1, _H), lambda i: (0, 0)),
            pl.BlockSpec((_H, _H), lambda i: (0, 0)),
            pl.BlockSpec((2, _R, 16), lambda i: (0, i, 0)),
        ],
        out_specs=pl.BlockSpec((_R, _H), lambda i: (i, 0)),
        out_shape=jax.ShapeDtypeStruct((_N, _H), jnp.float32),
    )(x, wp, bp, w1, degp)


def _tc2_body(aggp_ref, z1_ref, degp_ref, b1_ref, w2_ref, z2_ref):
    dinv = _deg_dinv(degp_ref)
    s = aggp_ref[0] + aggp_ref[1] - z1_ref[...]
    h1 = jnp.maximum(s * dinv[:, None] + b1_ref[...], 0.0)
    z2_ref[...] = jnp.dot(h1, w2_ref[...],
                          preferred_element_type=jnp.float32) * dinv[:, None]


def _tc2(aggp, z1, degp, b1, w2):
    return pl.pallas_call(
        _tc2_body,
        grid=(_N // _R,),
        in_specs=[
            pl.BlockSpec((2, _R, _H), lambda i: (0, i, 0)),
            pl.BlockSpec((_R, _H), lambda i: (i, 0)),
            pl.BlockSpec((2, _R, 16), lambda i: (0, i, 0)),
            pl.BlockSpec((1, _H), lambda i: (0, 0)),
            pl.BlockSpec((_H, _H), lambda i: (0, 0)),
        ],
        out_specs=pl.BlockSpec((_R, _H), lambda i: (i, 0)),
        out_shape=jax.ShapeDtypeStruct((_N, _H), jnp.float32),
    )(aggp, z1, degp, b1, w2)


def _tc3_body(aggp_ref, z2_ref, degp_ref, b2_ref, g_ref):
    i = pl.program_id(0)
    dinv = _deg_dinv(degp_ref)
    s = aggp_ref[0] + aggp_ref[1] - z2_ref[...]
    h2 = jnp.maximum(s * dinv[:, None] + b2_ref[...], 0.0)
    bsum = jnp.sum(h2, axis=0, keepdims=True)

    @pl.when(i == 0)
    def _():
        g_ref[...] = jnp.zeros_like(g_ref)

    g_ref[...] += bsum


def _tc3(aggp, z2, degp, b2):
    return pl.pallas_call(
        _tc3_body,
        grid=(_N // _R,),
        in_specs=[
            pl.BlockSpec((2, _R, _H), lambda i: (0, i, 0)),
            pl.BlockSpec((_R, _H), lambda i: (i, 0)),
            pl.BlockSpec((2, _R, 16), lambda i: (0, i, 0)),
            pl.BlockSpec((1, _H), lambda i: (0, 0)),
        ],
        out_specs=pl.BlockSpec((1, _H), lambda i: (0, 0)),
        out_shape=jax.ShapeDtypeStruct((1, _H), jnp.float32),
    )(aggp, z2, degp, b2)


def _tc4_body(g_ref, wc_ref, bc_ref, whl_ref, bhl_ref, wmlt_ref, bmlt_ref,
              wsl_ref, bsl_ref, wp1_ref, bp1_ref, wp2_ref, bp2_ref,
              wdt_ref, bdt_ref,
              value_ref, hl_ref, mlt_ref, p1_ref, p2_ref, dt_ref, sl_ref):
    j = pl.program_id(0)
    g = g_ref[...] * (1.0 / _N)

    @pl.when(j == 0)
    def _():
        value_ref[...] = jnp.dot(
            g, wc_ref[...], preferred_element_type=jnp.float32) + bc_ref[...]
        hl_ref[...] = jnp.dot(
            g, whl_ref[...], preferred_element_type=jnp.float32) + bhl_ref[...]
        mlt_ref[...] = jnp.dot(
            g, wmlt_ref[...],
            preferred_element_type=jnp.float32) + bmlt_ref[...]
        sl_ref[...] = jnp.dot(
            g, wsl_ref[...], preferred_element_type=jnp.float32) + bsl_ref[...]

    p1_ref[...] = jnp.dot(
        g, wp1_ref[...], preferred_element_type=jnp.float32) + bp1_ref[...]
    p2_ref[...] = jnp.dot(
        g, wp2_ref[...], preferred_element_type=jnp.float32) + bp2_ref[...]
    dt_ref[...] = jnp.dot(
        g, wdt_ref[...], preferred_element_type=jnp.float32) + bdt_ref[...]


def _tc4(gsum, p):
    small = lambda w: pl.BlockSpec(w, lambda j: (0, 0))
    bigw = pl.BlockSpec((_H, _CB), lambda j: (0, j))
    bigb = pl.BlockSpec((1, _CB), lambda j: (0, j))
    return pl.pallas_call(
        _tc4_body,
        grid=(pl.cdiv(_NUM_STATIONS, _CB),),
        in_specs=[
            small((1, _H)),
            small((_H, 1)), small((1, 1)),
            small((_H, 4)), small((1, 4)),
            small((_H, 3)), small((1, 3)),
            small((_H, 8)), small((1, 8)),
            bigw, bigb,
            bigw, bigb,
            bigw, bigb,
        ],
        out_specs=[
            small((1, 1)), small((1, 4)), small((1, 3)),
            pl.BlockSpec((1, _CB), lambda j: (0, j)),
            pl.BlockSpec((1, _CB), lambda j: (0, j)),
            pl.BlockSpec((1, _CB), lambda j: (0, j)),
            small((1, 8)),
        ],
        out_shape=[
            jax.ShapeDtypeStruct((1, 1), jnp.float32),
            jax.ShapeDtypeStruct((1, 4), jnp.float32),
            jax.ShapeDtypeStruct((1, 3), jnp.float32),
            jax.ShapeDtypeStruct((1, _NUM_STATIONS), jnp.float32),
            jax.ShapeDtypeStruct((1, _NUM_STATIONS), jnp.float32),
            jax.ShapeDtypeStruct((1, _NUM_STATIONS), jnp.float32),
            jax.ShapeDtypeStruct((1, 8), jnp.float32),
        ],
    )(gsum,
      p['W_critic'], p['b_critic'][None, :],
      p['W_hl'], p['b_hl'][None, :],
      p['W_mlt'], p['b_mlt'][None, :],
      p['W_sl'], p['b_sl'][None, :],
      p['W_p1'], p['b_p1'][None, :],
      p['W_p2'], p['b_p2'][None, :],
      p['W_dt'], p['b_dt'][None, :])


def kernel(node_features, edge_index, params):
    p = params
    ei = edge_index.astype(jnp.int32)
    e3 = ei.reshape(2, _NW, _NCH_W, _C).transpose(1, 2, 0, 3)
    dst2 = ei[1].reshape(_NW, _NCH_W, _C)
    ones16 = jnp.ones((_N, 16), jnp.float32)

    degp = _deg_kernel(ones16, dst2)
    z1 = _tc1(node_features, p['W_proj'], p['b_proj'][None, :], p['W1'], degp)
    agg1 = _edge_kernel(z1, e3)
    z2 = _tc2(agg1, z1, degp, p['b1'][None, :], p['W2'])
    agg2 = _edge_kernel(z2, e3)
    gsum = _tc3(agg2, z2, degp, p['b2'][None, :])
    value, hl, mlt, p1, p2, dt, sl = _tc4(gsum, p)
    return (value, hl, mlt, p1, p2, dt, sl)


# speedup vs baseline: 31.0562x; 1.3198x over previous
import functools

import jax
import jax.numpy as jnp
from jax import lax
from jax.experimental import pallas as pl
from jax.experimental.pallas import tpu as pltpu
from jax.experimental.pallas import tpu_sc as plsc

_N = 10000
_E = 320000
_H = 128
_NUM_STATIONS = 10000

_NW = 32
_C = 80
_NCH_W = _E // _NW // _C
_RPT = 624
_TAIL = _N - 16 * _RPT
_TAILBASE = 16 * _RPT

_R = 1000
_CB = 2048

_sc_mesh = plsc.VectorSubcoreMesh(core_axis_name="c", subcore_axis_name="s")


@functools.partial(
    pl.kernel,
    mesh=_sc_mesh,
    out_type=jax.ShapeDtypeStruct((2, _N, 16), jnp.float32),
    scratch_types=[
        pltpu.VMEM((_NCH_W, _C), jnp.int32),
        pltpu.VMEM((_C, 16), jnp.float32),
        pltpu.VMEM_SHARED((_N, 16), jnp.float32),
        pltpu.SemaphoreType.DMA,
    ],
)
def _deg_kernel(ones_hbm, dst_hbm, out_hbm, didx, onesv, acc, ssem):
    cid = lax.axis_index("c")
    sid = lax.axis_index("s")
    wid = cid * 16 + sid
    base = sid * _RPT
    pltpu.sync_copy(ones_hbm.at[pl.ds(base, _RPT)], acc.at[pl.ds(base, _RPT)])

    @pl.when(sid == 15)
    def _():
        pltpu.sync_copy(ones_hbm.at[pl.ds(_TAILBASE, _TAIL)],
                        acc.at[pl.ds(_TAILBASE, _TAIL)])

    pltpu.sync_copy(dst_hbm.at[wid], didx)
    pltpu.sync_copy(ones_hbm.at[pl.ds(0, _C)], onesv)
    plsc.subcore_barrier()

    def body(j, carry):
        pltpu.async_copy(onesv, acc.at[didx.at[j]], ssem, add=True)

        @pl.when(j >= 4)
        def _():
            pltpu.make_async_copy(onesv, acc.at[didx.at[j - 4]], ssem).wait()

        return carry

    lax.fori_loop(0, _NCH_W, body, 0)

    def drain(j, carry):
        pltpu.make_async_copy(onesv, acc.at[didx.at[j]], ssem).wait()
        return carry

    lax.fori_loop(_NCH_W - 4, _NCH_W, drain, 0)
    plsc.subcore_barrier()
    pltpu.sync_copy(acc.at[pl.ds(base, _RPT)],
                    out_hbm.at[cid, pl.ds(base, _RPT)])

    @pl.when(sid == 15)
    def _():
        pltpu.sync_copy(acc.at[pl.ds(_TAILBASE, _TAIL)],
                        out_hbm.at[cid, pl.ds(_TAILBASE, _TAIL)])


@functools.partial(
    pl.kernel,
    mesh=_sc_mesh,
    out_type=jax.ShapeDtypeStruct((2, _N, _H), jnp.float32),
    scratch_types=[
        pltpu.VMEM((6, 2, _C), jnp.int32),
        pltpu.VMEM((4, _C, _H), jnp.float32),
        pltpu.VMEM_SHARED((_N, _H), jnp.float32),
        pltpu.SemaphoreType.DMA,
        pltpu.SemaphoreType.DMA,
        pltpu.SemaphoreType.DMA,
    ],
)
def _edge_kernel(z_hbm, e_hbm, out_hbm, idx, rows, acc, isem, gsem, ssem):
    cid = lax.axis_index("c")
    sid = lax.axis_index("s")
    wid = cid * 16 + sid
    base = sid * _RPT
    pltpu.sync_copy(z_hbm.at[pl.ds(base, _RPT)], acc.at[pl.ds(base, _RPT)])

    @pl.when(sid == 15)
    def _():
        pltpu.sync_copy(z_hbm.at[pl.ds(_TAILBASE, _TAIL)],
                        acc.at[pl.ds(_TAILBASE, _TAIL)])

    plsc.subcore_barrier()

    for k in range(4):
        pltpu.async_copy(e_hbm.at[wid, k], idx.at[k], isem)
    for k in range(2):
        pltpu.make_async_copy(e_hbm.at[wid, k], idx.at[k], isem).wait()
        pltpu.async_copy(z_hbm.at[idx.at[k, 0]], rows.at[k], gsem)

    def body(j, carry):
        s4 = lax.rem(j, 4)
        s6 = lax.rem(j, 6)
        pltpu.make_async_copy(z_hbm.at[idx.at[s6, 0]], rows.at[s4],
                              gsem).wait()
        pltpu.async_copy(rows.at[s4], acc.at[idx.at[s6, 1]], ssem, add=True)

        @pl.when(j >= 2)
        def _():
            pltpu.make_async_copy(rows.at[lax.rem(j + 2, 4)],
                                  acc.at[idx.at[lax.rem(j + 4, 6), 1]],
                                  ssem).wait()

        @pl.when(j < _NCH_W - 2)
        def _():
            n6 = lax.rem(j + 2, 6)
            pltpu.make_async_copy(e_hbm.at[wid, j + 2], idx.at[n6],
                                  isem).wait()
            pltpu.async_copy(z_hbm.at[idx.at[n6, 0]], rows.at[lax.rem(j + 2, 4)],
                             gsem)

        @pl.when(j < _NCH_W - 4)
        def _():
            pltpu.async_copy(e_hbm.at[wid, j + 4], idx.at[lax.rem(j + 4, 6)],
                             isem)

        return carry

    lax.fori_loop(0, _NCH_W, body, 0)
    for k in (_NCH_W - 2, _NCH_W - 1):
        pltpu.make_async_copy(rows.at[k % 4], acc.at[idx.at[k % 6, 1]],
                              ssem).wait()
    plsc.subcore_barrier()
    pltpu.sync_copy(acc.at[pl.ds(base, _RPT)],
                    out_hbm.at[cid, pl.ds(base, _RPT)])

    @pl.when(sid == 15)
    def _():
        pltpu.sync_copy(acc.at[pl.ds(_TAILBASE, _TAIL)],
                        out_hbm.at[cid, pl.ds(_TAILBASE, _TAIL)])


def _deg_dinv(degp_ref):
    deg = (jnp.max(degp_ref[0], axis=-1) + jnp.max(degp_ref[1], axis=-1)
           - 1.0)
    return lax.rsqrt(deg)


def _tc1_body(x_ref, wp_ref, bp_ref, w1_ref, degp_ref, z_ref):
    h = jnp.maximum(
        jnp.dot(x_ref[...], wp_ref[...], preferred_element_type=jnp.float32)
        + bp_ref[...], 0.0)
    dinv = _deg_dinv(degp_ref)
    z_ref[...] = jnp.dot(h, w1_ref[...],
                         preferred_element_type=jnp.float32) * dinv[:, None]


def _tc1(x, wp, bp, w1, degp):
    return pl.pallas_call(
        _tc1_body,
        grid=(_N // _R,),
        in_specs=[
            pl.BlockSpec((_R, _H), lambda i: (i, 0)),
            pl.BlockSpec((_H, _H), lambda i: (0, 0)),
            pl.BlockSpec((1, _H), lambda i: (0, 0)),
            pl.BlockSpec((_H, _H), lambda i: (0, 0)),
            pl.BlockSpec((2, _R, 16), lambda i: (0, i, 0)),
        ],
        out_specs=pl.BlockSpec((_R, _H), lambda i: (i, 0)),
        out_shape=jax.ShapeDtypeStruct((_N, _H), jnp.float32),
    )(x, wp, bp, w1, degp)


def _tc2_body(aggp_ref, z1_ref, degp_ref, b1_ref, w2_ref, z2_ref):
    dinv = _deg_dinv(degp_ref)
    s = aggp_ref[0] + aggp_ref[1] - z1_ref[...]
    h1 = jnp.maximum(s * dinv[:, None] + b1_ref[...], 0.0)
    z2_ref[...] = jnp.dot(h1, w2_ref[...],
                          preferred_element_type=jnp.float32) * dinv[:, None]


def _tc2(aggp, z1, degp, b1, w2):
    return pl.pallas_call(
        _tc2_body,
        grid=(_N // _R,),
        in_specs=[
            pl.BlockSpec((2, _R, _H), lambda i: (0, i, 0)),
            pl.BlockSpec((_R, _H), lambda i: (i, 0)),
            pl.BlockSpec((2, _R, 16), lambda i: (0, i, 0)),
            pl.BlockSpec((1, _H), lambda i: (0, 0)),
            pl.BlockSpec((_H, _H), lambda i: (0, 0)),
        ],
        out_specs=pl.BlockSpec((_R, _H), lambda i: (i, 0)),
        out_shape=jax.ShapeDtypeStruct((_N, _H), jnp.float32),
    )(aggp, z1, degp, b1, w2)


def _tc34_body(aggp_ref, z2_ref, degp_ref, b2_ref,
               wc_ref, bc_ref, whl_ref, bhl_ref, wmlt_ref, bmlt_ref,
               wsl_ref, bsl_ref, wp1_ref, bp1_ref, wp2_ref, bp2_ref,
               wdt_ref, bdt_ref,
               value_ref, hl_ref, mlt_ref, p1_ref, p2_ref, dt_ref, sl_ref):
    i = pl.program_id(0)
    dinv = _deg_dinv(degp_ref)
    s = aggp_ref[0] + aggp_ref[1] - z2_ref[...]
    h2 = jnp.maximum(s * dinv[:, None] + b2_ref[...], 0.0)
    gpart = jnp.sum(h2, axis=0, keepdims=True) * (1.0 / _N)

    @pl.when(i == 0)
    def _():
        value_ref[...] = bc_ref[...]
        hl_ref[...] = bhl_ref[...]
        mlt_ref[...] = bmlt_ref[...]
        sl_ref[...] = bsl_ref[...]
        p1_ref[...] = bp1_ref[...]
        p2_ref[...] = bp2_ref[...]
        dt_ref[...] = bdt_ref[...]

    dot = lambda a, b: jnp.dot(a, b, preferred_element_type=jnp.float32)
    value_ref[...] += dot(gpart, wc_ref[...])
    hl_ref[...] += dot(gpart, whl_ref[...])
    mlt_ref[...] += dot(gpart, wmlt_ref[...])
    sl_ref[...] += dot(gpart, wsl_ref[...])
    p1_ref[...] += dot(gpart, wp1_ref[...])
    p2_ref[...] += dot(gpart, wp2_ref[...])
    dt_ref[...] += dot(gpart, wdt_ref[...])


def _tc34(aggp, z2, degp, b2, p):
    small = lambda w: pl.BlockSpec(w, lambda i: (0, 0))
    return pl.pallas_call(
        _tc34_body,
        grid=(_N // _R,),
        in_specs=[
            pl.BlockSpec((2, _R, _H), lambda i: (0, i, 0)),
            pl.BlockSpec((_R, _H), lambda i: (i, 0)),
            pl.BlockSpec((2, _R, 16), lambda i: (0, i, 0)),
            small((1, _H)),
            small((_H, 1)), small((1, 1)),
            small((_H, 4)), small((1, 4)),
            small((_H, 3)), small((1, 3)),
            small((_H, 8)), small((1, 8)),
            small((_H, _NUM_STATIONS)), small((1, _NUM_STATIONS)),
            small((_H, _NUM_STATIONS)), small((1, _NUM_STATIONS)),
            small((_H, _NUM_STATIONS)), small((1, _NUM_STATIONS)),
        ],
        out_specs=[
            small((1, 1)), small((1, 4)), small((1, 3)),
            small((1, _NUM_STATIONS)), small((1, _NUM_STATIONS)),
            small((1, _NUM_STATIONS)), small((1, 8)),
        ],
        out_shape=[
            jax.ShapeDtypeStruct((1, 1), jnp.float32),
            jax.ShapeDtypeStruct((1, 4), jnp.float32),
            jax.ShapeDtypeStruct((1, 3), jnp.float32),
            jax.ShapeDtypeStruct((1, _NUM_STATIONS), jnp.float32),
            jax.ShapeDtypeStruct((1, _NUM_STATIONS), jnp.float32),
            jax.ShapeDtypeStruct((1, _NUM_STATIONS), jnp.float32),
            jax.ShapeDtypeStruct((1, 8), jnp.float32),
        ],
    )(aggp, z2, degp, b2,
      p['W_critic'], p['b_critic'][None, :],
      p['W_hl'], p['b_hl'][None, :],
      p['W_mlt'], p['b_mlt'][None, :],
      p['W_sl'], p['b_sl'][None, :],
      p['W_p1'], p['b_p1'][None, :],
      p['W_p2'], p['b_p2'][None, :],
      p['W_dt'], p['b_dt'][None, :])


def kernel(node_features, edge_index, params):
    p = params
    ei = edge_index.astype(jnp.int32)
    e3 = ei.reshape(2, _NW, _NCH_W, _C).transpose(1, 2, 0, 3)
    dst2 = ei[1].reshape(_NW, _NCH_W, _C)
    ones16 = jnp.ones((_N, 16), jnp.float32)

    degp = _deg_kernel(ones16, dst2)
    z1 = _tc1(node_features, p['W_proj'], p['b_proj'][None, :], p['W1'], degp)
    agg1 = _edge_kernel(z1, e3)
    z2 = _tc2(agg1, z1, degp, p['b1'][None, :], p['W2'])
    agg2 = _edge_kernel(z2, e3)
    value, hl, mlt, p1, p2, dt, sl = _tc34(agg2, z2, degp,
                                           p['b2'][None, :], p)
    return (value, hl, mlt, p1, p2, dt, sl)
